# baseline (speedup 1.0000x reference)
import functools

import jax
import jax.numpy as jnp
from jax import lax
from jax.experimental import pallas as pl
from jax.experimental.pallas import tpu as pltpu
from jax.experimental.pallas import tpu_sc as plsc

N = 1000000
B = 16384
NC = 2
NS = 16
NW = NC * NS
BPW = B // NW
CHUNK = 128
NQCH = BPW // CHUNK
NTCH = 2 * BPW // CHUNK
STEPS = BPW // 16


def _body(rots_hbm, trans_flat_hbm, idx_hbm, tidx_hbm, rot_out, tran_out,
          idx_v, tidx_v, q_v, t_v, rot_v, sem):
    c = lax.axis_index("c")
    s = lax.axis_index("s")
    wid = s * NC + c
    base = wid * BPW

    pltpu.sync_copy(idx_hbm.at[wid], idx_v)
    pltpu.sync_copy(tidx_hbm.at[wid], tidx_v)
    copies = []
    for j in range(NQCH):
        sl = pl.ds(j * CHUNK, CHUNK)
        copies.append(pltpu.async_copy(rots_hbm.at[idx_v.at[j]], q_v.at[sl], sem))
    for j in range(NTCH):
        sl = pl.ds(j * CHUNK, CHUNK)
        copies.append(
            pltpu.async_copy(trans_flat_hbm.at[tidx_v.at[j]], t_v.at[sl], sem))
    for cp in copies:
        cp.wait()

    lanes = lax.iota(jnp.int32, 16)

    def step(st, carry):
        row = st * 16 + lanes

        def comp(col):
            cols = jnp.full((16,), col, jnp.int32)
            return plsc.load_gather(q_v, [row, cols])

        qr = comp(0)
        qi = comp(1)
        qj = comp(2)
        qk = comp(3)

        inv = 2.0 / (qr * qr + qi * qi + qj * qj + qk * qk)
        ii = qi * qi * inv
        jj = qj * qj * inv
        kk = qk * qk * inv
        ij = qi * qj * inv
        ik = qi * qk * inv
        jk = qj * qk * inv
        ir = qi * qr * inv
        jr = qj * qr * inv
        kr = qk * qr * inv

        def put(col, val):
            cols = jnp.full((16,), col, jnp.int32)
            plsc.store_scatter(rot_v, [row, cols], val)

        put(0, 1.0 - (jj + kk))
        put(1, ij - kr)
        put(2, ik + jr)
        put(3, ij + kr)
        put(4, 1.0 - (ii + kk))
        put(5, jk - ir)
        put(6, ik - jr)
        put(7, jk + ir)
        put(8, 1.0 - (ii + jj))
        return carry

    lax.fori_loop(0, STEPS, step, 0)

    pltpu.sync_copy(rot_v, rot_out.at[pl.ds(base, BPW)])
    pltpu.sync_copy(t_v, tran_out.at[pl.ds(2 * base, 2 * BPW)])


_sc_call = functools.partial(
    pl.kernel,
    out_type=(
        jax.ShapeDtypeStruct((B, 9), jnp.float32),
        jax.ShapeDtypeStruct((2 * B,), jnp.float32),
    ),
    mesh=plsc.VectorSubcoreMesh(core_axis_name="c", subcore_axis_name="s"),
    compiler_params=pltpu.CompilerParams(
        needs_layout_passes=False, use_tc_tiling_on_sc=False
    ),
    scratch_types=[
        pltpu.VMEM((NQCH, CHUNK), jnp.int32),
        pltpu.VMEM((NTCH, CHUNK), jnp.int32),
        pltpu.VMEM((BPW, 4), jnp.float32),
        pltpu.VMEM((2 * BPW,), jnp.float32),
        pltpu.VMEM((BPW, 9), jnp.float32),
        pltpu.SemaphoreType.DMA,
    ],
)(_body)


@jax.jit
def kernel(rots_emb_w, trans_emb_w, ind):
    ind32 = ind.astype(jnp.int32)
    idx = ind32.reshape(NW, NQCH, CHUNK)
    tidx = (2 * ind32[:, None] + jnp.arange(2, dtype=jnp.int32)).reshape(
        NW, NTCH, CHUNK)
    rot9, tran_flat = _sc_call(
        rots_emb_w, trans_emb_w.reshape(2 * N), idx, tidx)
    return rot9.reshape(B, 3, 3), tran_flat.reshape(B, 2)


# speedup vs baseline: 13.4550x; 13.4550x over previous
import functools

import jax
import jax.numpy as jnp
from jax import lax
from jax.experimental import pallas as pl
from jax.experimental.pallas import tpu as pltpu
from jax.experimental.pallas import tpu_sc as plsc

N = 1000000
B = 16384
NC = 2
NS = 16
NW = NC * NS
BPW = B // NW
CHUNK = 128
NQCH = 4 * BPW // CHUNK
NTCH = 2 * BPW // CHUNK
STEPS = BPW // 16
VPAD = 1000064
NT = VPAD // 128


def _body(qflat_hbm, tflat_hbm, qaddr_hbm, taddr_hbm, rot_out, tran_out,
          qa_v, ta_v, q_v, t_v, rot_v, sem):
    c = lax.axis_index("c")
    s = lax.axis_index("s")
    wid = s * NC + c
    base = wid * BPW

    pltpu.sync_copy(qaddr_hbm.at[wid], qa_v)
    pltpu.sync_copy(taddr_hbm.at[wid], ta_v)
    copies = []
    for j in range(NQCH):
        sl = pl.ds(j * CHUNK, CHUNK)
        copies.append(pltpu.async_copy(qflat_hbm.at[qa_v.at[j]], q_v.at[sl], sem))
    for j in range(NTCH):
        sl = pl.ds(j * CHUNK, CHUNK)
        copies.append(pltpu.async_copy(tflat_hbm.at[ta_v.at[j]], t_v.at[sl], sem))
    for cp in copies:
        cp.wait()

    lanes = lax.iota(jnp.int32, 16)

    def step(st, carry):
        row4 = (st * 16 + lanes) * 4

        def comp(col):
            return plsc.load_gather(q_v, [row4 + col])

        qr = comp(0)
        qi = comp(1)
        qj = comp(2)
        qk = comp(3)

        inv = 2.0 / (qr * qr + qi * qi + qj * qj + qk * qk)
        ii = qi * qi * inv
        jj = qj * qj * inv
        kk = qk * qk * inv
        ij = qi * qj * inv
        ik = qi * qk * inv
        jk = qj * qk * inv
        ir = qi * qr * inv
        jr = qj * qr * inv
        kr = qk * qr * inv

        row9 = (st * 16 + lanes) * 9

        def put(col, val):
            plsc.store_scatter(rot_v, [row9 + col], val)

        put(0, 1.0 - (jj + kk))
        put(1, ij - kr)
        put(2, ik + jr)
        put(3, ij + kr)
        put(4, 1.0 - (ii + kk))
        put(5, jk - ir)
        put(6, ik - jr)
        put(7, jk + ir)
        put(8, 1.0 - (ii + jj))
        return carry

    lax.fori_loop(0, STEPS, step, 0)

    pltpu.sync_copy(rot_v, rot_out.at[pl.ds(9 * base, 9 * BPW)])
    pltpu.sync_copy(t_v, tran_out.at[pl.ds(2 * base, 2 * BPW)])


_sc_call = functools.partial(
    pl.kernel,
    out_type=(
        jax.ShapeDtypeStruct((9 * B,), jnp.float32),
        jax.ShapeDtypeStruct((2 * B,), jnp.float32),
    ),
    mesh=plsc.VectorSubcoreMesh(core_axis_name="c", subcore_axis_name="s"),
    compiler_params=pltpu.CompilerParams(
        needs_layout_passes=False, use_tc_tiling_on_sc=False
    ),
    scratch_types=[
        pltpu.VMEM((NQCH, CHUNK), jnp.int32),
        pltpu.VMEM((NTCH, CHUNK), jnp.int32),
        pltpu.VMEM((4 * BPW,), jnp.float32),
        pltpu.VMEM((2 * BPW,), jnp.float32),
        pltpu.VMEM((9 * BPW,), jnp.float32),
        pltpu.SemaphoreType.DMA,
    ],
)(_body)


def _tile_view(table, ncomp):
    padded = jnp.pad(table, ((0, VPAD - N), (0, 0)))
    return (padded.T.reshape(ncomp, NT, 128)
            .transpose(1, 0, 2)
            .reshape(NT * ncomp * 128))


@jax.jit
def kernel(rots_emb_w, trans_emb_w, ind):
    ind32 = ind.astype(jnp.int32)
    tile = ind32 >> 7
    lane = ind32 & 127
    qaddr = ((tile * 512 + lane)[:, None]
             + jnp.arange(0, 512, 128, dtype=jnp.int32)).reshape(NW, NQCH, CHUNK)
    taddr = ((tile * 256 + lane)[:, None]
             + jnp.arange(0, 256, 128, dtype=jnp.int32)).reshape(NW, NTCH, CHUNK)
    rot9, tran_flat = _sc_call(
        _tile_view(rots_emb_w, 4), _tile_view(trans_emb_w, 2), qaddr, taddr)
    return rot9.reshape(B, 3, 3), tran_flat.reshape(B, 2)


# speedup vs baseline: 22.0324x; 1.6375x over previous
import functools

import jax
import jax.numpy as jnp
from jax import lax
from jax.experimental import pallas as pl
from jax.experimental.pallas import tpu as pltpu
from jax.experimental.pallas import tpu_sc as plsc

N = 1000000
B = 16384
NC = 2
NS = 16
NW = NC * NS
BPW = B // NW
CHUNK = 64
NCH = BPW // CHUNK
VPAD = 1000064
NT = VPAD // 128


def _body(q3_hbm, t3_hbm, tile_hbm, lane_hbm, rot_out, tran_out,
          tile_v, lane_v, qb0, qb1, tb0, tb1, rot_v, tout_v, sem):
    c = lax.axis_index("c")
    s = lax.axis_index("s")
    wid = s * NC + c
    base = wid * BPW

    pltpu.sync_copy(tile_hbm.at[wid], tile_v)
    pltpu.sync_copy(lane_hbm.at[wid], lane_v)

    qbufs = (qb0, qb1)
    tbufs = (tb0, tb1)
    lanes = lax.iota(jnp.int32, 16)

    def fire(j, buf):
        qcp = pltpu.async_copy(q3_hbm.at[tile_v.at[j]], qbufs[buf], sem)
        tcp = pltpu.async_copy(t3_hbm.at[tile_v.at[j]], tbufs[buf], sem)
        return qcp, tcp

    def drain(cps):
        for cp in cps:
            cp.wait()

    def extract(j, buf):
        qb = qbufs[buf]
        tb = tbufs[buf]
        for g in range(CHUNK // 16):
            i0 = g * 16
            slot = i0 + lanes
            lane = plsc.load_gather(lane_v, [jnp.full((16,), j, jnp.int32),
                                             slot])
            row = (j * CHUNK + i0) + lanes

            def compq(col):
                return plsc.load_gather(
                    qb, [slot, jnp.full((16,), col, jnp.int32), lane])

            qr = compq(0)
            qi = compq(1)
            qj = compq(2)
            qk = compq(3)

            inv = 2.0 / (qr * qr + qi * qi + qj * qj + qk * qk)
            ii = qi * qi * inv
            jj = qj * qj * inv
            kk = qk * qk * inv
            ij = qi * qj * inv
            ik = qi * qk * inv
            jk = qj * qk * inv
            ir = qi * qr * inv
            jr = qj * qr * inv
            kr = qk * qr * inv

            def put(col, val):
                plsc.store_scatter(
                    rot_v, [row, jnp.full((16,), col, jnp.int32)], val)

            put(0, 1.0 - (jj + kk))
            put(1, ij - kr)
            put(2, ik + jr)
            put(3, ij + kr)
            put(4, 1.0 - (ii + kk))
            put(5, jk - ir)
            put(6, ik - jr)
            put(7, jk + ir)
            put(8, 1.0 - (ii + jj))

            for col in range(2):
                cols = jnp.full((16,), col, jnp.int32)
                tval = plsc.load_gather(tb, [slot, cols, lane])
                plsc.store_scatter(tout_v, [row, cols], tval)

    cps = fire(0, 0)
    for j in range(NCH):
        drain(cps)
        if j + 1 < NCH:
            cps = fire(j + 1, (j + 1) % 2)
        extract(j, j % 2)

    pltpu.sync_copy(rot_v, rot_out.at[pl.ds(base, BPW)])
    pltpu.sync_copy(tout_v, tran_out.at[pl.ds(base, BPW)])


_sc_call = functools.partial(
    pl.kernel,
    out_type=(
        jax.ShapeDtypeStruct((B, 9), jnp.float32),
        jax.ShapeDtypeStruct((B, 2), jnp.float32),
    ),
    mesh=plsc.VectorSubcoreMesh(core_axis_name="c", subcore_axis_name="s"),
    compiler_params=pltpu.CompilerParams(
        needs_layout_passes=False, use_tc_tiling_on_sc=False
    ),
    scratch_types=[
        pltpu.VMEM((NCH, CHUNK), jnp.int32),
        pltpu.VMEM((NCH, CHUNK), jnp.int32),
        pltpu.VMEM((CHUNK, 4, 128), jnp.float32),
        pltpu.VMEM((CHUNK, 4, 128), jnp.float32),
        pltpu.VMEM((CHUNK, 2, 128), jnp.float32),
        pltpu.VMEM((CHUNK, 2, 128), jnp.float32),
        pltpu.VMEM((BPW, 9), jnp.float32),
        pltpu.VMEM((BPW, 2), jnp.float32),
        pltpu.SemaphoreType.DMA,
    ],
)(_body)


def _tile_view(table, ncomp):
    padded = jnp.pad(table, ((0, VPAD - N), (0, 0)))
    return padded.T.reshape(ncomp, NT, 128).transpose(1, 0, 2)


@jax.jit
def kernel(rots_emb_w, trans_emb_w, ind):
    ind32 = ind.astype(jnp.int32)
    tile = (ind32 >> 7).reshape(NW, NCH, CHUNK)
    lane = (ind32 & 127).reshape(NW, NCH, CHUNK)
    rot9, tran = _sc_call(
        _tile_view(rots_emb_w, 4), _tile_view(trans_emb_w, 2), tile, lane)
    return rot9.reshape(B, 3, 3), tran


# speedup vs baseline: 24.8473x; 1.1278x over previous
import functools

import jax
import jax.numpy as jnp
from jax import lax
from jax.experimental import pallas as pl
from jax.experimental.pallas import tpu as pltpu
from jax.experimental.pallas import tpu_sc as plsc

N = 1000000
B = 16384
NC = 2
NS = 16
NW = NC * NS
BPW = B // NW
CHUNK = 64
NCH = BPW // CHUNK
VPAD = 1000064
NT = VPAD // 128


def _worker_id_and_idx(tile_hbm, lane_hbm, tile_v, lane_v):
    c = lax.axis_index("c")
    s = lax.axis_index("s")
    wid = s * NC + c
    pltpu.sync_copy(tile_hbm.at[wid], tile_v)
    pltpu.sync_copy(lane_hbm.at[wid], lane_v)
    return wid


def _q_body(q3_hbm, tile_hbm, lane_hbm, rot_out,
            tile_v, lane_v, qb0, qb1, rot_v, sem):
    wid = _worker_id_and_idx(tile_hbm, lane_hbm, tile_v, lane_v)
    base = wid * BPW
    qbufs = (qb0, qb1)
    lanes = lax.iota(jnp.int32, 16)

    def fire(j, buf):
        return (pltpu.async_copy(q3_hbm.at[tile_v.at[j]], qbufs[buf], sem),)

    def extract(j, buf):
        qb = qbufs[buf]
        for g in range(CHUNK // 16):
            i0 = g * 16
            slot = i0 + lanes
            lane = plsc.load_gather(lane_v, [jnp.full((16,), j, jnp.int32),
                                             slot])
            row = (j * CHUNK + i0) + lanes

            def compq(col):
                return plsc.load_gather(
                    qb, [slot, jnp.full((16,), col, jnp.int32), lane])

            qr = compq(0)
            qi = compq(1)
            qj = compq(2)
            qk = compq(3)

            inv = 2.0 / (qr * qr + qi * qi + qj * qj + qk * qk)
            ii = qi * qi * inv
            jj = qj * qj * inv
            kk = qk * qk * inv
            ij = qi * qj * inv
            ik = qi * qk * inv
            jk = qj * qk * inv
            ir = qi * qr * inv
            jr = qj * qr * inv
            kr = qk * qr * inv

            def put(col, val):
                plsc.store_scatter(
                    rot_v, [row, jnp.full((16,), col, jnp.int32)], val)

            put(0, 1.0 - (jj + kk))
            put(1, ij - kr)
            put(2, ik + jr)
            put(3, ij + kr)
            put(4, 1.0 - (ii + kk))
            put(5, jk - ir)
            put(6, ik - jr)
            put(7, jk + ir)
            put(8, 1.0 - (ii + jj))

    cps = fire(0, 0)
    for j in range(NCH):
        for cp in cps:
            cp.wait()
        if j + 1 < NCH:
            cps = fire(j + 1, (j + 1) % 2)
        extract(j, j % 2)

    pltpu.sync_copy(rot_v, rot_out.at[pl.ds(base, BPW)])


def _t_body(t3_hbm, tile_hbm, lane_hbm, tran_out,
            tile_v, lane_v, tb0, tb1, tout_v, sem):
    wid = _worker_id_and_idx(tile_hbm, lane_hbm, tile_v, lane_v)
    base = wid * BPW
    tbufs = (tb0, tb1)
    lanes = lax.iota(jnp.int32, 16)

    def fire(j, buf):
        return (pltpu.async_copy(t3_hbm.at[tile_v.at[j]], tbufs[buf], sem),)

    def extract(j, buf):
        tb = tbufs[buf]
        for g in range(CHUNK // 16):
            i0 = g * 16
            slot = i0 + lanes
            lane = plsc.load_gather(lane_v, [jnp.full((16,), j, jnp.int32),
                                             slot])
            row = (j * CHUNK + i0) + lanes
            for col in range(2):
                cols = jnp.full((16,), col, jnp.int32)
                tval = plsc.load_gather(tb, [slot, cols, lane])
                plsc.store_scatter(tout_v, [row, cols], tval)

    cps = fire(0, 0)
    for j in range(NCH):
        for cp in cps:
            cp.wait()
        if j + 1 < NCH:
            cps = fire(j + 1, (j + 1) % 2)
        extract(j, j % 2)

    pltpu.sync_copy(tout_v, tran_out.at[pl.ds(base, BPW)])


_MESH = plsc.VectorSubcoreMesh(core_axis_name="c", subcore_axis_name="s")
_PARAMS = pltpu.CompilerParams(
    needs_layout_passes=False, use_tc_tiling_on_sc=False)

_q_call = functools.partial(
    pl.kernel,
    out_type=jax.ShapeDtypeStruct((B, 9), jnp.float32),
    mesh=_MESH,
    compiler_params=_PARAMS,
    scratch_types=[
        pltpu.VMEM((NCH, CHUNK), jnp.int32),
        pltpu.VMEM((NCH, CHUNK), jnp.int32),
        pltpu.VMEM((CHUNK, 4, 128), jnp.float32),
        pltpu.VMEM((CHUNK, 4, 128), jnp.float32),
        pltpu.VMEM((BPW, 9), jnp.float32),
        pltpu.SemaphoreType.DMA,
    ],
)(_q_body)

_t_call = functools.partial(
    pl.kernel,
    out_type=jax.ShapeDtypeStruct((B, 2), jnp.float32),
    mesh=_MESH,
    compiler_params=_PARAMS,
    scratch_types=[
        pltpu.VMEM((NCH, CHUNK), jnp.int32),
        pltpu.VMEM((NCH, CHUNK), jnp.int32),
        pltpu.VMEM((CHUNK, 2, 128), jnp.float32),
        pltpu.VMEM((CHUNK, 2, 128), jnp.float32),
        pltpu.VMEM((BPW, 2), jnp.float32),
        pltpu.SemaphoreType.DMA,
    ],
)(_t_body)


def _tile_view(table, ncomp):
    padded = jnp.pad(table, ((0, VPAD - N), (0, 0)))
    return padded.T.reshape(ncomp, NT, 128).transpose(1, 0, 2)


@jax.jit
def kernel(rots_emb_w, trans_emb_w, ind):
    ind32 = ind.astype(jnp.int32)
    tile = (ind32 >> 7).reshape(NW, NCH, CHUNK)
    lane = (ind32 & 127).reshape(NW, NCH, CHUNK)
    tran = _t_call(_tile_view(trans_emb_w, 2), tile, lane)
    rot9 = _q_call(_tile_view(rots_emb_w, 4), tile, lane)
    return rot9.reshape(B, 3, 3), tran
